# baseline (speedup 1.0000x reference)
import jax
import jax.numpy as jnp
from jax.experimental import pallas as pl
from jax.experimental.pallas import tpu as pltpu

N = 100000
D = 128
CHUNK = 5000
NBUF = 4
NCHUNKS = N // CHUNK


def _gcn_pipe(x_hbm, adj_ref, w_ref, b_ref, o_hbm, xbuf, obuf, m_ref, rsem, wsem):
    m_ref[...] = jax.lax.dot_general(
        w_ref[...], adj_ref[...],
        dimension_numbers=(((1,), (1,)), ((), ())),
        preferred_element_type=jnp.float32,
    )

    def rcopy(i, slot):
        return pltpu.make_async_copy(
            x_hbm.at[pl.ds(i * CHUNK, CHUNK), :], xbuf.at[slot], rsem.at[slot]
        )

    def wcopy(i, slot):
        return pltpu.make_async_copy(
            obuf.at[slot], o_hbm.at[pl.ds(i * CHUNK, CHUNK), :], wsem.at[slot]
        )

    for i in range(NBUF):
        rcopy(i, i).start()

    bias = b_ref[...]
    for i in range(NCHUNKS):
        slot = i % NBUF
        rcopy(i, slot).wait()
        if i >= NBUF:
            wcopy(i - NBUF, slot).wait()
        obuf[slot] = (
            jnp.dot(xbuf[slot], m_ref[...], preferred_element_type=jnp.float32)
            + bias
        )
        wcopy(i, slot).start()
        if i + NBUF < NCHUNKS:
            rcopy(i + NBUF, slot).start()

    for i in range(NCHUNKS - NBUF, NCHUNKS):
        wcopy(i, i % NBUF).wait()


def kernel(input, adj, weight, bias):
    bias2d = bias.reshape(1, D)
    return pl.pallas_call(
        _gcn_pipe,
        in_specs=[
            pl.BlockSpec(memory_space=pltpu.MemorySpace.HBM),
            pl.BlockSpec((D, D), lambda: (0, 0)),
            pl.BlockSpec((D, D), lambda: (0, 0)),
            pl.BlockSpec((1, D), lambda: (0, 0)),
        ],
        out_specs=pl.BlockSpec(memory_space=pltpu.MemorySpace.HBM),
        out_shape=jax.ShapeDtypeStruct((N, D), jnp.float32),
        scratch_shapes=[
            pltpu.VMEM((NBUF, CHUNK, D), jnp.float32),
            pltpu.VMEM((NBUF, CHUNK, D), jnp.float32),
            pltpu.VMEM((D, D), jnp.float32),
            pltpu.SemaphoreType.DMA((NBUF,)),
            pltpu.SemaphoreType.DMA((NBUF,)),
        ],
    )(input, adj, weight, bias2d)


# speedup vs baseline: 1.0534x; 1.0534x over previous
import jax
import jax.numpy as jnp
from jax.experimental import pallas as pl
from jax.experimental.pallas import tpu as pltpu

N = 100000
D = 128
CHUNK = 10000
NBUF = 3
NCHUNKS = N // CHUNK


def _gcn_pipe(x_hbm, adj_ref, w_ref, b_ref, o_hbm, xbuf, obuf, m_ref, rsem, wsem):
    def rcopy(i, slot):
        return pltpu.make_async_copy(
            x_hbm.at[pl.ds(i * CHUNK, CHUNK), :], xbuf.at[slot], rsem.at[slot]
        )

    def wcopy(i, slot):
        return pltpu.make_async_copy(
            obuf.at[slot], o_hbm.at[pl.ds(i * CHUNK, CHUNK), :], wsem.at[slot]
        )

    for i in range(NBUF):
        rcopy(i, i).start()

    m_ref[...] = jax.lax.dot_general(
        w_ref[...], adj_ref[...],
        dimension_numbers=(((1,), (1,)), ((), ())),
        preferred_element_type=jnp.float32,
    )

    bias = b_ref[...]
    for i in range(NCHUNKS):
        slot = i % NBUF
        rcopy(i, slot).wait()
        if i >= NBUF:
            wcopy(i - NBUF, slot).wait()
        obuf[slot] = (
            jnp.dot(xbuf[slot], m_ref[...], preferred_element_type=jnp.float32)
            + bias
        )
        wcopy(i, slot).start()
        if i + NBUF < NCHUNKS:
            rcopy(i + NBUF, slot).start()

    for i in range(NCHUNKS - NBUF, NCHUNKS):
        wcopy(i, i % NBUF).wait()


def kernel(input, adj, weight, bias):
    bias2d = bias.reshape(1, D)
    return pl.pallas_call(
        _gcn_pipe,
        in_specs=[
            pl.BlockSpec(memory_space=pltpu.MemorySpace.HBM),
            pl.BlockSpec((D, D), lambda: (0, 0)),
            pl.BlockSpec((D, D), lambda: (0, 0)),
            pl.BlockSpec((1, D), lambda: (0, 0)),
        ],
        out_specs=pl.BlockSpec(memory_space=pltpu.MemorySpace.HBM),
        out_shape=jax.ShapeDtypeStruct((N, D), jnp.float32),
        scratch_shapes=[
            pltpu.VMEM((NBUF, CHUNK, D), jnp.float32),
            pltpu.VMEM((NBUF, CHUNK, D), jnp.float32),
            pltpu.VMEM((D, D), jnp.float32),
            pltpu.SemaphoreType.DMA((NBUF,)),
            pltpu.SemaphoreType.DMA((NBUF,)),
        ],
    )(input, adj, weight, bias2d)
